# baseline (speedup 1.0000x reference)
import jax
import jax.numpy as jnp
from jax import lax
from jax.experimental import pallas as pl
from jax.experimental.pallas import tpu as pltpu
from jax.experimental.pallas import tpu_sc as plsc

FROZEN = 500_000
DIM = 64
NC, NS = 2, 16
NW = NC * NS
L = 16
G = 128
KB = 4
BR = G * KB


def _body(idx_hbm, p_hbm, t_hbm, out_hbm,
          idx_v, pidx, pdst, tidx, tdst, rowbuf, gsem, ssem):
    n_total = idx_hbm.shape[0]
    chunk = n_total // NW
    wid = lax.axis_index("s") * NC + lax.axis_index("c")
    base = wid * chunk
    pltpu.sync_copy(idx_hbm.at[pl.ds(base, chunk)], idx_v)

    lane = lax.iota(jnp.int32, L)
    zeros = jnp.zeros((L,), jnp.int32)

    def compact_step(i, carry):
        cnt_p, cnt_t = carry
        v = idx_v[pl.ds(i * L, L)]
        pos_g = base + i * L + lane
        m_p = v < FROZEN
        mp_i = m_p.astype(jnp.int32)
        pp = cnt_p + plsc.cumsum(mp_i) - mp_i
        plsc.store_scatter(pidx, [pp], v, mask=m_p)
        plsc.store_scatter(pdst, [pp], pos_g, mask=m_p)
        m_t = jnp.logical_not(m_p)
        mt_i = 1 - mp_i
        tp = cnt_t + plsc.cumsum(mt_i) - mt_i
        plsc.store_scatter(tidx, [tp], v - FROZEN, mask=m_t)
        plsc.store_scatter(tdst, [tp], pos_g, mask=m_t)
        cnt_p = cnt_p + plsc.all_reduce_population_count(m_p)
        cnt_t = cnt_t + plsc.all_reduce_population_count(m_t)
        return cnt_p, cnt_t

    cnt_p, cnt_t = lax.fori_loop(0, chunk // L, compact_step, (zeros, zeros))

    def pad_list(ilist, dlist, cnt_vec):
        count = cnt_vec[0]
        first_i = plsc.load_gather(ilist, [zeros])
        first_d = plsc.load_gather(dlist, [zeros])
        end = ((count + (BR - 1)) >> 9) << 9
        for j in range(BR // L):
            pos = count + j * L + lane
            mm = pos < end
            plsc.store_scatter(ilist, [pos], first_i, mask=mm)
            plsc.store_scatter(dlist, [pos], first_d, mask=mm)
        return end >> 9

    def run_list(ilist, dlist, n_b, table):
        def batch_step(b, _):
            hs = [pltpu.async_copy(table.at[ilist.at[pl.ds((b * KB + k) * G, G)]],
                                   rowbuf.at[pl.ds(k * G, G)], gsem)
                  for k in range(KB)]
            for h in hs:
                h.wait()
            hs2 = [pltpu.async_copy(rowbuf.at[pl.ds(k * G, G)],
                                    out_hbm.at[dlist.at[pl.ds((b * KB + k) * G, G)]], ssem)
                   for k in range(KB)]
            for h in hs2:
                h.wait()
            return 0
        lax.fori_loop(0, n_b, batch_step, 0)

    run_list(pidx, pdst, pad_list(pidx, pdst, cnt_p), p_hbm)
    run_list(tidx, tdst, pad_list(tidx, tdst, cnt_t), t_hbm)


def _tc_transpose(table):
    v = table.shape[0]
    blk = 8192
    grid = (v + blk - 1) // blk

    def tbody(in_ref, out_ref):
        out_ref[...] = in_ref[...].T

    return pl.pallas_call(
        tbody,
        grid=(grid,),
        in_specs=[pl.BlockSpec((DIM, blk), lambda i: (0, i))],
        out_specs=pl.BlockSpec((blk, DIM), lambda i: (i, 0)),
        out_shape=jax.ShapeDtypeStruct((v, DIM), jnp.float32),
    )(table.T)


def kernel(batch, pretrained_weight, trainable_weight):
    flat = batch.reshape(-1).astype(jnp.int32)
    n = flat.shape[0]
    chunk = n // NW
    lcap = chunk + BR
    out = pl.kernel(
        _body,
        out_type=jax.ShapeDtypeStruct((n, DIM), jnp.float32),
        mesh=plsc.VectorSubcoreMesh(core_axis_name="c", subcore_axis_name="s"),
        compiler_params=pltpu.CompilerParams(needs_layout_passes=False, use_tc_tiling_on_sc=False),
        scratch_types=[
            pltpu.VMEM((chunk,), jnp.int32),
            pltpu.VMEM((lcap,), jnp.int32),
            pltpu.VMEM((lcap,), jnp.int32),
            pltpu.VMEM((lcap,), jnp.int32),
            pltpu.VMEM((lcap,), jnp.int32),
            pltpu.VMEM((BR, DIM), jnp.float32),
            pltpu.SemaphoreType.DMA,
            pltpu.SemaphoreType.DMA,
        ],
    )(flat, pretrained_weight, _tc_transpose(trainable_weight))
    return out.reshape(batch.shape + (DIM,))


# speedup vs baseline: 1.0792x; 1.0792x over previous
import jax
import jax.numpy as jnp
from jax import lax
from jax.experimental import pallas as pl
from jax.experimental.pallas import tpu as pltpu
from jax.experimental.pallas import tpu_sc as plsc

FROZEN = 500_000
DIM = 64
NC, NS = 2, 16
NW = NC * NS
L = 16
G = 128
KB = 4
BR = G * KB


def _body(idx_hbm, p_hbm, t_hbm, out_hbm,
          idx_v, pidx, pdst, tidx, tdst, rowbuf, gsem, ssem):
    n_total = idx_hbm.shape[0]
    chunk = n_total // NW
    wid = lax.axis_index("s") * NC + lax.axis_index("c")
    base = wid * chunk
    pltpu.sync_copy(idx_hbm.at[pl.ds(base, chunk)], idx_v)

    lane = lax.iota(jnp.int32, L)
    zeros = jnp.zeros((L,), jnp.int32)

    def compact_step(i, carry):
        cnt_p, cnt_t = carry
        v = idx_v[pl.ds(i * L, L)]
        pos_g = base + i * L + lane
        m_p = v < FROZEN
        mp_i = m_p.astype(jnp.int32)
        pp = cnt_p + plsc.cumsum(mp_i) - mp_i
        plsc.store_scatter(pidx, [pp], v, mask=m_p)
        plsc.store_scatter(pdst, [pp], pos_g, mask=m_p)
        m_t = jnp.logical_not(m_p)
        mt_i = 1 - mp_i
        tp = cnt_t + plsc.cumsum(mt_i) - mt_i
        plsc.store_scatter(tidx, [tp], v - FROZEN, mask=m_t)
        plsc.store_scatter(tdst, [tp], pos_g, mask=m_t)
        cnt_p = cnt_p + plsc.all_reduce_population_count(m_p)
        cnt_t = cnt_t + plsc.all_reduce_population_count(m_t)
        return cnt_p, cnt_t

    cnt_p, cnt_t = lax.fori_loop(0, chunk // L, compact_step, (zeros, zeros))

    def pad_list(ilist, dlist, cnt_vec):
        count = cnt_vec[0]
        first_i = plsc.load_gather(ilist, [zeros])
        first_d = plsc.load_gather(dlist, [zeros])
        end = ((count + (BR - 1)) >> 9) << 9
        for j in range(BR // L):
            pos = count + j * L + lane
            mm = pos < end
            plsc.store_scatter(ilist, [pos], first_i, mask=mm)
            plsc.store_scatter(dlist, [pos], first_d, mask=mm)
        return end >> 9

    def run_list(ilist, dlist, n_b, table):
        def batch_step(b, _):
            hs = [pltpu.async_copy(table.at[ilist.at[pl.ds((b * KB + k) * G, G)]],
                                   rowbuf.at[pl.ds(k * G, G)], gsem)
                  for k in range(KB)]
            for h in hs:
                h.wait()
            hs2 = [pltpu.async_copy(rowbuf.at[pl.ds(k * G, G)],
                                    out_hbm.at[dlist.at[pl.ds((b * KB + k) * G, G)]], ssem)
                   for k in range(KB)]
            for h in hs2:
                h.wait()
            return 0
        lax.fori_loop(0, n_b, batch_step, 0)

    run_list(pidx, pdst, pad_list(pidx, pdst, cnt_p), p_hbm)
    run_list(tidx, tdst, pad_list(tidx, tdst, cnt_t), t_hbm)


def kernel(batch, pretrained_weight, trainable_weight):
    flat = batch.reshape(-1).astype(jnp.int32)
    n = flat.shape[0]
    chunk = n // NW
    lcap = chunk + BR
    out = pl.kernel(
        _body,
        out_type=jax.ShapeDtypeStruct((n, DIM), jnp.float32),
        mesh=plsc.VectorSubcoreMesh(core_axis_name="c", subcore_axis_name="s"),
        compiler_params=pltpu.CompilerParams(needs_layout_passes=False, use_tc_tiling_on_sc=False),
        scratch_types=[
            pltpu.VMEM((chunk,), jnp.int32),
            pltpu.VMEM((lcap,), jnp.int32),
            pltpu.VMEM((lcap,), jnp.int32),
            pltpu.VMEM((lcap,), jnp.int32),
            pltpu.VMEM((lcap,), jnp.int32),
            pltpu.VMEM((BR, DIM), jnp.float32),
            pltpu.SemaphoreType.DMA,
            pltpu.SemaphoreType.DMA,
        ],
    )(flat, pretrained_weight, trainable_weight)
    return out.reshape(batch.shape + (DIM,))
